# Initial kernel scaffold; baseline (speedup 1.0000x reference)
import jax
import jax.numpy as jnp
from jax.experimental import pallas as pl


def kernel(x, actions, tar_scores, geo, wall_batch, category, batch, edge_index, params1, params2):
    raise NotImplementedError("write your pallas kernel here")


# speedup vs baseline: 2.4503x; 2.4503x over previous
import functools
import jax
import jax.numpy as jnp
from jax import lax
from jax.experimental import pallas as pl
from jax.experimental.pallas import tpu as pltpu
from jax.experimental.pallas import tpu_sc as plsc

N = 10000
E = 320000
NW = 32
TN = 1000
TE = 4000
S1_C = 80
S2_C = 2000

f32 = jnp.float32


def _dot(a, b):
    return jax.lax.dot_general(a, b, (((1,), (0,)), ((), ())),
                               preferred_element_type=f32)


def _t0_body(in10_ref, geo_ref, wall_ref, cat_ref, bat_ref,
             ew_ref, cw_ref, iw_ref, gw_ref, ww_ref, dw_ref, sw_ref,
             a_ref, b_ref, cond_ref):
    in10 = in10_ref[...]
    geo = geo_ref[...]
    wb = wall_ref[...]
    cat = cat_ref[0, 0, :]
    bat = bat_ref[0, 0, :]

    cat_oh = (cat[:, None] ==
              lax.broadcasted_iota(jnp.int32, (TN, 10), 1)).astype(f32)
    bat_oh = (bat[:, None] ==
              lax.broadcasted_iota(jnp.int32, (TN, 100), 1)).astype(f32)

    for net in range(2):
        emb = ew_ref[net]
        cW, cb = cw_ref[net, 0], cw_ref[net, 1]
        ctab = _dot(jnp.tanh(emb), cW)
        cf = jnp.tanh(_dot(cat_oh, ctab) + cb[0:1, :])

        w1, b1 = ww_ref[net, 0, 0:1, :], ww_ref[net, 1, 0:1, :]
        w2, b2 = ww_ref[net, 2, :, :], ww_ref[net, 3, 0:1, :]
        wtab = jnp.tanh(_dot(jnp.tanh(wb * w1 + b1), w2) + b2)
        wf = _dot(bat_oh, wtab)

        g1, gb1 = gw_ref[net, 0, 0:8, :], gw_ref[net, 1, 0:1, :]
        g2, gb2 = gw_ref[net, 2, :, :], gw_ref[net, 3, 0:1, :]
        gf = jnp.tanh(_dot(jnp.tanh(_dot(geo, g1) + gb1), g2) + gb2)

        i1, ib1 = iw_ref[net, 0, 0:16, :], iw_ref[net, 1, 0:1, :]
        i2, ib2 = iw_ref[net, 2, :, :], iw_ref[net, 3, 0:1, :]
        h0 = jnp.tanh(_dot(jnp.tanh(_dot(in10, i1) + ib1), i2) + ib2)

        cond = jnp.concatenate([cf, wf, gf], axis=-1)
        g1cat = jnp.concatenate([h0, cond], axis=-1)

        wd, bd = dw_ref[net, :, :], dw_ref[net + 2, 0:1, 0:64]
        ws = sw_ref[net, :, :]
        a_ref[:, net * 64:(net + 1) * 64] = _dot(g1cat, wd) + bd
        b_ref[:, net * 64:(net + 1) * 64] = _dot(g1cat, ws)
        cond_ref[:, net * 96:(net + 1) * 96] = cond


def _t0(in10, geo, wall, cat3, bat3, ew, cw, iw, gw, ww, dw, sw):
    g = N // TN
    full = lambda s: pl.BlockSpec(s, lambda i: tuple(0 for _ in s))
    return pl.pallas_call(
        _t0_body,
        grid=(g,),
        in_specs=[
            pl.BlockSpec((TN, 16), lambda i: (i, 0)),
            pl.BlockSpec((TN, 8), lambda i: (i, 0)),
            full((100, 1)),
            pl.BlockSpec((1, 1, TN), lambda i: (i, 0, 0)),
            pl.BlockSpec((1, 1, TN), lambda i: (i, 0, 0)),
            full(ew.shape), full(cw.shape), full(iw.shape),
            full(gw.shape), full(ww.shape), full(dw.shape), full(sw.shape),
        ],
        out_specs=[
            pl.BlockSpec((TN, 128), lambda i: (i, 0)),
            pl.BlockSpec((TN, 128), lambda i: (i, 0)),
            pl.BlockSpec((TN, 192), lambda i: (i, 0)),
        ],
        out_shape=[
            jax.ShapeDtypeStruct((N, 128), f32),
            jax.ShapeDtypeStruct((N, 128), f32),
            jax.ShapeDtypeStruct((N, 192), f32),
        ],
    )(in10, geo, wall, cat3, bat3, ew, cw, iw, gw, ww, dw, sw)


def _s1_body(a_hbm, b_hbm, src_hbm, dst_hbm, p_hbm,
             dstv, srcv, ga, gb, sema, semb):
    wid = lax.axis_index("s") * 2 + lax.axis_index("c")
    per_w = E // NW

    def chunk(k, _):
        base = pl.multiple_of(wid * per_w + k * S1_C, 16)
        pltpu.sync_copy(dst_hbm.at[pl.ds(base, S1_C)], dstv)
        pltpu.sync_copy(src_hbm.at[pl.ds(base, S1_C)], srcv)
        cpa = pltpu.async_copy(a_hbm.at[dstv], ga, sema)
        cpb = pltpu.async_copy(b_hbm.at[srcv], gb, semb)
        cpa.wait()
        cpb.wait()

        def addrow(r, _):
            for cg in range(8):
                s = pl.ds(cg * 16, 16)
                ga[r, s] = ga[r, s] + gb[r, s]
            return 0
        lax.fori_loop(0, S1_C, addrow, 0)
        pltpu.sync_copy(ga, p_hbm.at[pl.ds(base, S1_C), :])
        return 0
    lax.fori_loop(0, per_w // S1_C, chunk, 0)


def _s1(a, b, src, dst):
    mesh = plsc.VectorSubcoreMesh(core_axis_name="c", subcore_axis_name="s")
    k = functools.partial(
        pl.kernel, mesh=mesh,
        compiler_params=pltpu.CompilerParams(use_tc_tiling_on_sc=False, needs_layout_passes=False),
        out_type=jax.ShapeDtypeStruct((E, 128), f32),
        scratch_types=[
            pltpu.VMEM((S1_C,), jnp.int32),
            pltpu.VMEM((S1_C,), jnp.int32),
            pltpu.VMEM((S1_C, 128), f32),
            pltpu.VMEM((S1_C, 128), f32),
            pltpu.SemaphoreType.DMA,
            pltpu.SemaphoreType.DMA,
        ],
    )(_s1_body)
    return k(a, b, src, dst)


def _t1_body(p_ref, w_ref, b_ref, m_ref):
    t = jnp.tanh(p_ref[...])
    m_ref[...] = _dot(t, w_ref[...]) + b_ref[0:1, :]


def _t1(p, w2blk, b2cat):
    g = E // TE
    return pl.pallas_call(
        _t1_body,
        grid=(g,),
        in_specs=[
            pl.BlockSpec((TE, 128), lambda i: (i, 0)),
            pl.BlockSpec((128, 128), lambda i: (0, 0)),
            pl.BlockSpec((1, 128), lambda i: (0, 0)),
        ],
        out_specs=pl.BlockSpec((TE, 128), lambda i: (i, 0)),
        out_shape=jax.ShapeDtypeStruct((E, 128), f32),
    )(p, w2blk, b2cat)


def _s2_body(m_hbm, dst_hbm, neg_hbm, o_hbm, acc, mbuf, dbuf):
    wid = lax.axis_index("s") * 2 + lax.axis_index("c")
    cg = wid % 16
    h = wid // 16
    half = E // 2
    pltpu.sync_copy(neg_hbm, acc)
    i16 = lax.iota(jnp.int32, 16)
    hi = i16 >> 3
    col = i16 & 7

    def chunk(k, _):
        base = pl.multiple_of(h * half + k * S2_C, 16)
        pltpu.sync_copy(dst_hbm.at[pl.ds(base, S2_C)], dbuf)
        pltpu.sync_copy(m_hbm.at[pl.ds(base, S2_C), pl.ds(cg * 8, 8)], mbuf)

        def pair(p, _):
            ri = 2 * p + hi
            dv = plsc.load_gather(dbuf, [ri])
            dw = plsc.load_gather(dbuf, [ri ^ 1])
            mv = plsc.load_gather(mbuf, [ri, col])
            mw = plsc.load_gather(mbuf, [ri ^ 1, col])
            mv = jnp.where(dv == dw, jnp.maximum(mv, mw), mv)
            old = plsc.load_gather(acc, [dv, col])
            plsc.store_scatter(acc, [dv, col], jnp.maximum(old, mv))
            return 0
        lax.fori_loop(0, S2_C // 2, pair, 0)
        return 0
    lax.fori_loop(0, half // S2_C, chunk, 0)
    pltpu.sync_copy(acc, o_hbm.at[wid])


def _s2(m, dst, neg):
    mesh = plsc.VectorSubcoreMesh(core_axis_name="c", subcore_axis_name="s")
    k = functools.partial(
        pl.kernel, mesh=mesh,
        compiler_params=pltpu.CompilerParams(use_tc_tiling_on_sc=False, needs_layout_passes=False),
        out_type=jax.ShapeDtypeStruct((NW, N, 8), f32),
        scratch_types=[
            pltpu.VMEM((N, 8), f32),
            pltpu.VMEM((S2_C, 8), f32),
            pltpu.VMEM((S2_C,), jnp.int32),
        ],
    )(_s2_body)
    return k(m, dst, neg)


def _strips(o):
    out = []
    for cg in range(16):
        v = jnp.maximum(o[cg], o[cg + 16])
        v = jnp.where(jnp.isfinite(v), v, 0.0)
        out.append(jnp.tanh(v))
    return out


def _tmid_body(o_ref, cond_ref, dw_ref, sw_ref, a_ref, b_ref):
    s = _strips(o_ref[...])
    cond = cond_ref[...]
    for net in range(2):
        cn = cond[:, net * 96:(net + 1) * 96]
        wd, bd = dw_ref[net, :, :], dw_ref[net + 2, 0:1, 0:64]
        ws = sw_ref[net, :, :]
        a = _dot(cn, wd[64:160, :]) + bd
        b = _dot(cn, ws[64:160, :])
        for j in range(8):
            st = s[net * 8 + j]
            a = a + _dot(st, wd[8 * j:8 * j + 8, :])
            b = b + _dot(st, ws[8 * j:8 * j + 8, :])
        a_ref[:, net * 64:(net + 1) * 64] = a
        b_ref[:, net * 64:(net + 1) * 64] = b


def _tmid(o, cond, dw, sw):
    g = N // TN
    return pl.pallas_call(
        _tmid_body,
        grid=(g,),
        in_specs=[
            pl.BlockSpec((NW, TN, 8), lambda i: (0, i, 0)),
            pl.BlockSpec((TN, 192), lambda i: (i, 0)),
            pl.BlockSpec(dw.shape, lambda i: (0, 0, 0)),
            pl.BlockSpec(sw.shape, lambda i: (0, 0, 0)),
        ],
        out_specs=[
            pl.BlockSpec((TN, 128), lambda i: (i, 0)),
            pl.BlockSpec((TN, 128), lambda i: (i, 0)),
        ],
        out_shape=[
            jax.ShapeDtypeStruct((N, 128), f32),
            jax.ShapeDtypeStruct((N, 128), f32),
        ],
    )(o, cond, dw, sw)


def _ttail_body(o_ref, cond_ref, tw_ref, fw_ref, q1_ref, q2_ref):
    s = _strips(o_ref[...])
    cond = cond_ref[...]
    for net in range(2):
        cn = cond[:, net * 96:(net + 1) * 96]
        w3, b3 = tw_ref[net, :, :], tw_ref[net + 2, 0:1, 0:64]
        t = _dot(cn, w3[64:160, :]) + b3
        for j in range(8):
            t = t + _dot(s[net * 8 + j], w3[8 * j:8 * j + 8, :])
        t = jnp.tanh(t)
        w4, b4 = fw_ref[net, :, :], fw_ref[net + 2, 0:1, :]
        q = _dot(t, w4) + b4
        if net == 0:
            q1_ref[...] = q
        else:
            q2_ref[...] = q


def _ttail(o, cond, tw, fw):
    g = N // TN
    return pl.pallas_call(
        _ttail_body,
        grid=(g,),
        in_specs=[
            pl.BlockSpec((NW, TN, 8), lambda i: (0, i, 0)),
            pl.BlockSpec((TN, 192), lambda i: (i, 0)),
            pl.BlockSpec(tw.shape, lambda i: (0, 0, 0)),
            pl.BlockSpec(fw.shape, lambda i: (0, 0, 0)),
        ],
        out_specs=[
            pl.BlockSpec((TN, 8), lambda i: (i, 0)),
            pl.BlockSpec((TN, 8), lambda i: (i, 0)),
        ],
        out_shape=[
            jax.ShapeDtypeStruct((N, 8), f32),
            jax.ShapeDtypeStruct((N, 8), f32),
        ],
    )(o, cond, tw, fw)


def kernel(x, actions, tar_scores, geo, wall_batch, category, batch,
           edge_index, params1, params2):
    in10 = jnp.concatenate([x, actions, tar_scores], axis=-1)
    in10 = jnp.pad(in10, ((0, 0), (0, 6)))
    geo8 = jnp.pad(geo.astype(f32), ((0, 0), (0, 6)))
    cat3 = category.astype(jnp.int32).reshape(N // TN, 1, TN)
    bat3 = batch.astype(jnp.int32).reshape(N // TN, 1, TN)
    src = edge_index[0].astype(jnp.int32)
    dst = edge_index[1].astype(jnp.int32)
    neg = jnp.full((N, 8), -jnp.inf, f32)

    ps = (params1, params2)
    ew = jnp.stack([p['embed_table'] for p in ps])
    cw = jnp.stack([jnp.stack([p['embed_lin'][0],
                               jnp.broadcast_to(p['embed_lin'][1], (32, 32))])
                    for p in ps])

    def pack2(key, d1pad):
        mats = []
        for p in ps:
            (w1, b1), (w2, b2) = p[key]
            d = w1.shape[0]
            w1p = jnp.zeros((d1pad, w1.shape[1]), f32).at[:d, :].set(w1)
            w2p = jnp.zeros((d1pad, w2.shape[1]), f32).at[:w2.shape[0], :].set(w2)
            b1p = jnp.broadcast_to(b1, (d1pad, w1.shape[1]))
            b2p = jnp.broadcast_to(b2, (d1pad, w2.shape[1]))
            mats.append(jnp.stack([w1p, b1p, w2p, b2p]))
        return jnp.stack(mats)

    iw = pack2('init_enc', 64)
    gw = pack2('geo_enc', 32)
    ww = pack2('wall_enc', 32)

    def split_conv(key):
        wds, wss, bds = [], [], []
        for p in ps:
            (w1, b1), _ = p[key]
            wds.append(w1[:160] - w1[160:])
            wss.append(w1[160:])
            bds.append(jnp.broadcast_to(b1, (160, 64)))
        return (jnp.stack(wds + bds), jnp.stack(wss + bds))

    dw1, sw1 = split_conv('mlp1')
    dw2, sw2 = split_conv('mlp2')

    def blk(key):
        w = jnp.zeros((128, 128), f32)
        w = w.at[:64, :64].set(ps[0][key][1][0])
        w = w.at[64:, 64:].set(ps[1][key][1][0])
        b = jnp.concatenate([ps[0][key][1][1], ps[1][key][1][1]])
        return w, b.reshape(1, 128)

    w2blk1, b2cat1 = blk('mlp1')
    w2blk2, b2cat2 = blk('mlp2')

    tws, fws = [], []
    for p in ps:
        (w3, b3), (w4, b4) = p['tail']
        tws.append(w3)
        fws.append(jnp.zeros((64, 8), f32).at[:, 0].set(w4[:, 0]))
    tw = jnp.stack(tws + [jnp.broadcast_to(b3i, (160, 64))
                          for b3i in [ps[0]['tail'][0][1], ps[1]['tail'][0][1]]])
    fw = jnp.stack(fws + [jnp.zeros((64, 8), f32).at[:, 0].set(p['tail'][1][1][0])
                          for p in ps])

    ww_in = wall_batch.astype(f32)

    a1, b1arr, cond = _t0(in10, geo8, ww_in, cat3, bat3,
                          ew, cw, iw, gw, ww, dw1, sw1)
    p1 = _s1(a1, b1arr, src, dst)
    m1 = _t1(p1, w2blk1, b2cat1)
    o1 = _s2(m1, dst, neg)
    a2, b2arr = _tmid(o1, cond, dw2, sw2)
    p2 = _s1(a2, b2arr, src, dst)
    m2 = _t1(p2, w2blk2, b2cat2)
    o2 = _s2(m2, dst, neg)
    q1p, q2p = _ttail(o2, cond, tw, fw)
    return (q1p[:, :1], q2p[:, :1])


# speedup vs baseline: 2.4799x; 1.0121x over previous
import functools
import jax
import jax.numpy as jnp
from jax import lax
from jax.experimental import pallas as pl
from jax.experimental.pallas import tpu as pltpu
from jax.experimental.pallas import tpu_sc as plsc

N = 10000
E = 320000
NW = 32
TN = 1000
TE = 4000
S1_C = 80
S2_C = 2000

f32 = jnp.float32


def _dot(a, b):
    return jax.lax.dot_general(a, b, (((1,), (0,)), ((), ())),
                               preferred_element_type=f32)


def _t0_body(in10_ref, geo_ref, wall_ref, cat_ref, bat_ref,
             ew_ref, cw_ref, iw_ref, gw_ref, ww_ref, dw_ref, sw_ref,
             a_ref, b_ref, cond_ref):
    in10 = in10_ref[...]
    geo = geo_ref[...]
    wb = wall_ref[...]
    cat = cat_ref[0, 0, :]
    bat = bat_ref[0, 0, :]

    cat_oh = (cat[:, None] ==
              lax.broadcasted_iota(jnp.int32, (TN, 10), 1)).astype(f32)
    bat_oh = (bat[:, None] ==
              lax.broadcasted_iota(jnp.int32, (TN, 100), 1)).astype(f32)

    for net in range(2):
        emb = ew_ref[net]
        cW, cb = cw_ref[net, 0], cw_ref[net, 1]
        ctab = _dot(jnp.tanh(emb), cW)
        cf = jnp.tanh(_dot(cat_oh, ctab) + cb[0:1, :])

        w1, b1 = ww_ref[net, 0, 0:1, :], ww_ref[net, 1, 0:1, :]
        w2, b2 = ww_ref[net, 2, :, :], ww_ref[net, 3, 0:1, :]
        wtab = jnp.tanh(_dot(jnp.tanh(wb * w1 + b1), w2) + b2)
        wf = _dot(bat_oh, wtab)

        g1, gb1 = gw_ref[net, 0, 0:8, :], gw_ref[net, 1, 0:1, :]
        g2, gb2 = gw_ref[net, 2, :, :], gw_ref[net, 3, 0:1, :]
        gf = jnp.tanh(_dot(jnp.tanh(_dot(geo, g1) + gb1), g2) + gb2)

        i1, ib1 = iw_ref[net, 0, 0:16, :], iw_ref[net, 1, 0:1, :]
        i2, ib2 = iw_ref[net, 2, :, :], iw_ref[net, 3, 0:1, :]
        h0 = jnp.tanh(_dot(jnp.tanh(_dot(in10, i1) + ib1), i2) + ib2)

        cond = jnp.concatenate([cf, wf, gf], axis=-1)
        g1cat = jnp.concatenate([h0, cond], axis=-1)

        wd, bd = dw_ref[net, :, :], dw_ref[net + 2, 0:1, 0:64]
        ws = sw_ref[net, :, :]
        a_ref[:, net * 64:(net + 1) * 64] = _dot(g1cat, wd) + bd
        b_ref[:, net * 64:(net + 1) * 64] = _dot(g1cat, ws)
        cond_ref[:, net * 96:(net + 1) * 96] = cond


def _t0(in10, geo, wall, cat3, bat3, ew, cw, iw, gw, ww, dw, sw):
    g = N // TN
    full = lambda s: pl.BlockSpec(s, lambda i: tuple(0 for _ in s))
    return pl.pallas_call(
        _t0_body,
        grid=(g,),
        in_specs=[
            pl.BlockSpec((TN, 16), lambda i: (i, 0)),
            pl.BlockSpec((TN, 8), lambda i: (i, 0)),
            full((100, 1)),
            pl.BlockSpec((1, 1, TN), lambda i: (i, 0, 0)),
            pl.BlockSpec((1, 1, TN), lambda i: (i, 0, 0)),
            full(ew.shape), full(cw.shape), full(iw.shape),
            full(gw.shape), full(ww.shape), full(dw.shape), full(sw.shape),
        ],
        out_specs=[
            pl.BlockSpec((TN, 128), lambda i: (i, 0)),
            pl.BlockSpec((TN, 128), lambda i: (i, 0)),
            pl.BlockSpec((TN, 192), lambda i: (i, 0)),
        ],
        out_shape=[
            jax.ShapeDtypeStruct((N, 128), f32),
            jax.ShapeDtypeStruct((N, 128), f32),
            jax.ShapeDtypeStruct((N, 192), f32),
        ],
    )(in10, geo, wall, cat3, bat3, ew, cw, iw, gw, ww, dw, sw)


def _s1_body(a_hbm, b_hbm, src_hbm, dst_hbm, p_hbm,
             dstv0, srcv0, ga0, gb0, dstv1, srcv1, ga1, gb1,
             sg0, sg1, sw0, sw1):
    wid = lax.axis_index("s") * 2 + lax.axis_index("c")
    per_w = E // NW
    nch = per_w // S1_C
    slots = ((dstv0, srcv0, ga0, gb0, sg0, sw0),
             (dstv1, srcv1, ga1, gb1, sg1, sw1))

    def fetch(c, b):
        dv, sv, ga, gb, sg, _ = slots[b]
        base = pl.multiple_of(wid * per_w + c * S1_C, 16)
        pltpu.sync_copy(dst_hbm.at[pl.ds(base, S1_C)], dv)
        pltpu.sync_copy(src_hbm.at[pl.ds(base, S1_C)], sv)
        pltpu.async_copy(a_hbm.at[dv], ga, sg)
        pltpu.async_copy(b_hbm.at[sv], gb, sg)

    def drain_g(b):
        _, _, ga, gb, sg, _ = slots[b]
        pltpu.make_async_copy(a_hbm.at[pl.ds(0, S1_C)], ga, sg).wait()
        pltpu.make_async_copy(b_hbm.at[pl.ds(0, S1_C)], gb, sg).wait()

    def process(c, b):
        _, _, ga, gb, _, sw = slots[b]
        drain_g(b)

        def addrow(r, _):
            for cg in range(8):
                s = pl.ds(cg * 16, 16)
                ga[r, s] = ga[r, s] + gb[r, s]
            return 0
        lax.fori_loop(0, S1_C, addrow, 0)
        base = pl.multiple_of(wid * per_w + c * S1_C, 16)
        pltpu.async_copy(ga, p_hbm.at[pl.ds(base, S1_C), :], sw)

    def wait_w(b):
        _, _, ga, _, _, sw = slots[b]
        pltpu.make_async_copy(ga, p_hbm.at[pl.ds(0, S1_C), :], sw).wait()

    fetch(0, 0)

    def pair(i, _):
        c = 2 * i

        @pl.when(c + 1 < nch)
        def _():
            @pl.when(c > 0)
            def _():
                wait_w(1)
            fetch(c + 1, 1)
        process(c, 0)

        @pl.when(c + 2 < nch)
        def _():
            wait_w(0)
            fetch(c + 2, 0)

        @pl.when(c + 1 < nch)
        def _():
            process(c + 1, 1)
        return 0
    lax.fori_loop(0, (nch + 1) // 2, pair, 0)
    wait_w(0)

    @pl.when(nch > 1)
    def _():
        wait_w(1)


def _s1(a, b, src, dst):
    mesh = plsc.VectorSubcoreMesh(core_axis_name="c", subcore_axis_name="s")
    k = functools.partial(
        pl.kernel, mesh=mesh,
        compiler_params=pltpu.CompilerParams(use_tc_tiling_on_sc=False, needs_layout_passes=False),
        out_type=jax.ShapeDtypeStruct((E, 128), f32),
        scratch_types=[
            pltpu.VMEM((S1_C,), jnp.int32),
            pltpu.VMEM((S1_C,), jnp.int32),
            pltpu.VMEM((S1_C, 128), f32),
            pltpu.VMEM((S1_C, 128), f32),
            pltpu.VMEM((S1_C,), jnp.int32),
            pltpu.VMEM((S1_C,), jnp.int32),
            pltpu.VMEM((S1_C, 128), f32),
            pltpu.VMEM((S1_C, 128), f32),
            pltpu.SemaphoreType.DMA,
            pltpu.SemaphoreType.DMA,
            pltpu.SemaphoreType.DMA,
            pltpu.SemaphoreType.DMA,
        ],
    )(_s1_body)
    return k(a, b, src, dst)


def _t1_body(p_ref, w_ref, b_ref, m_ref):
    t = jnp.tanh(p_ref[...])
    m_ref[...] = _dot(t, w_ref[...]) + b_ref[0:1, :]


def _t1(p, w2blk, b2cat):
    g = E // TE
    return pl.pallas_call(
        _t1_body,
        grid=(g,),
        in_specs=[
            pl.BlockSpec((TE, 128), lambda i: (i, 0)),
            pl.BlockSpec((128, 128), lambda i: (0, 0)),
            pl.BlockSpec((1, 128), lambda i: (0, 0)),
        ],
        out_specs=pl.BlockSpec((TE, 128), lambda i: (i, 0)),
        out_shape=jax.ShapeDtypeStruct((E, 128), f32),
    )(p, w2blk, b2cat)


def _s2_body(m_hbm, dst_hbm, neg_hbm, o_hbm, acc, mbuf, dbuf, stage):
    wid = lax.axis_index("s") * 2 + lax.axis_index("c")
    cg = wid % 16
    h = wid // 16
    half = E // 2
    pltpu.sync_copy(neg_hbm, acc)
    i16 = lax.iota(jnp.int32, 16)
    hi = i16 >> 3
    col = i16 & 7

    def chunk(k, _):
        base = pl.multiple_of(h * half + k * S2_C, 16)
        pltpu.sync_copy(dst_hbm.at[pl.ds(base, S2_C)], dbuf)
        pltpu.sync_copy(m_hbm.at[pl.ds(base, S2_C), pl.ds(cg * 8, 8)], mbuf)

        @plsc.parallel_loop(0, S2_C // 2, unroll=4)
        def premax(p):
            ri = 2 * p + hi
            dv = plsc.load_gather(dbuf, [ri])
            dw = plsc.load_gather(dbuf, [ri ^ 1])
            mv = plsc.load_gather(mbuf, [ri, col])
            mw = plsc.load_gather(mbuf, [ri ^ 1, col])
            stage[p, :] = jnp.where(dv == dw, jnp.maximum(mv, mw), mv)

        def pair(p, _):
            ri = 2 * p + hi
            dv = plsc.load_gather(dbuf, [ri])
            old = plsc.load_gather(acc, [dv, col])
            plsc.store_scatter(acc, [dv, col], jnp.maximum(old, stage[p, :]))
            return 0
        lax.fori_loop(0, S2_C // 2, pair, 0)
        return 0
    lax.fori_loop(0, half // S2_C, chunk, 0)
    pltpu.sync_copy(acc, o_hbm.at[wid])


def _s2(m, dst, neg):
    mesh = plsc.VectorSubcoreMesh(core_axis_name="c", subcore_axis_name="s")
    k = functools.partial(
        pl.kernel, mesh=mesh,
        compiler_params=pltpu.CompilerParams(use_tc_tiling_on_sc=False, needs_layout_passes=False),
        out_type=jax.ShapeDtypeStruct((NW, N, 8), f32),
        scratch_types=[
            pltpu.VMEM((N, 8), f32),
            pltpu.VMEM((S2_C, 8), f32),
            pltpu.VMEM((S2_C,), jnp.int32),
            pltpu.VMEM((S2_C // 2, 16), f32),
        ],
    )(_s2_body)
    return k(m, dst, neg)


def _strips(o):
    out = []
    for cg in range(16):
        v = jnp.maximum(o[cg], o[cg + 16])
        v = jnp.where(jnp.isfinite(v), v, 0.0)
        out.append(jnp.tanh(v))
    return out


def _tmid_body(o_ref, cond_ref, dw_ref, sw_ref, a_ref, b_ref):
    s = _strips(o_ref[...])
    cond = cond_ref[...]
    for net in range(2):
        cn = cond[:, net * 96:(net + 1) * 96]
        wd, bd = dw_ref[net, :, :], dw_ref[net + 2, 0:1, 0:64]
        ws = sw_ref[net, :, :]
        a = _dot(cn, wd[64:160, :]) + bd
        b = _dot(cn, ws[64:160, :])
        for j in range(8):
            st = s[net * 8 + j]
            a = a + _dot(st, wd[8 * j:8 * j + 8, :])
            b = b + _dot(st, ws[8 * j:8 * j + 8, :])
        a_ref[:, net * 64:(net + 1) * 64] = a
        b_ref[:, net * 64:(net + 1) * 64] = b


def _tmid(o, cond, dw, sw):
    g = N // TN
    return pl.pallas_call(
        _tmid_body,
        grid=(g,),
        in_specs=[
            pl.BlockSpec((NW, TN, 8), lambda i: (0, i, 0)),
            pl.BlockSpec((TN, 192), lambda i: (i, 0)),
            pl.BlockSpec(dw.shape, lambda i: (0, 0, 0)),
            pl.BlockSpec(sw.shape, lambda i: (0, 0, 0)),
        ],
        out_specs=[
            pl.BlockSpec((TN, 128), lambda i: (i, 0)),
            pl.BlockSpec((TN, 128), lambda i: (i, 0)),
        ],
        out_shape=[
            jax.ShapeDtypeStruct((N, 128), f32),
            jax.ShapeDtypeStruct((N, 128), f32),
        ],
    )(o, cond, dw, sw)


def _ttail_body(o_ref, cond_ref, tw_ref, fw_ref, q1_ref, q2_ref):
    s = _strips(o_ref[...])
    cond = cond_ref[...]
    for net in range(2):
        cn = cond[:, net * 96:(net + 1) * 96]
        w3, b3 = tw_ref[net, :, :], tw_ref[net + 2, 0:1, 0:64]
        t = _dot(cn, w3[64:160, :]) + b3
        for j in range(8):
            t = t + _dot(s[net * 8 + j], w3[8 * j:8 * j + 8, :])
        t = jnp.tanh(t)
        w4, b4 = fw_ref[net, :, :], fw_ref[net + 2, 0:1, :]
        q = _dot(t, w4) + b4
        if net == 0:
            q1_ref[...] = q
        else:
            q2_ref[...] = q


def _ttail(o, cond, tw, fw):
    g = N // TN
    return pl.pallas_call(
        _ttail_body,
        grid=(g,),
        in_specs=[
            pl.BlockSpec((NW, TN, 8), lambda i: (0, i, 0)),
            pl.BlockSpec((TN, 192), lambda i: (i, 0)),
            pl.BlockSpec(tw.shape, lambda i: (0, 0, 0)),
            pl.BlockSpec(fw.shape, lambda i: (0, 0, 0)),
        ],
        out_specs=[
            pl.BlockSpec((TN, 8), lambda i: (i, 0)),
            pl.BlockSpec((TN, 8), lambda i: (i, 0)),
        ],
        out_shape=[
            jax.ShapeDtypeStruct((N, 8), f32),
            jax.ShapeDtypeStruct((N, 8), f32),
        ],
    )(o, cond, tw, fw)


def kernel(x, actions, tar_scores, geo, wall_batch, category, batch,
           edge_index, params1, params2):
    in10 = jnp.concatenate([x, actions, tar_scores], axis=-1)
    in10 = jnp.pad(in10, ((0, 0), (0, 6)))
    geo8 = jnp.pad(geo.astype(f32), ((0, 0), (0, 6)))
    cat3 = category.astype(jnp.int32).reshape(N // TN, 1, TN)
    bat3 = batch.astype(jnp.int32).reshape(N // TN, 1, TN)
    src = edge_index[0].astype(jnp.int32)
    dst = edge_index[1].astype(jnp.int32)
    neg = jnp.full((N, 8), -jnp.inf, f32)

    ps = (params1, params2)
    ew = jnp.stack([p['embed_table'] for p in ps])
    cw = jnp.stack([jnp.stack([p['embed_lin'][0],
                               jnp.broadcast_to(p['embed_lin'][1], (32, 32))])
                    for p in ps])

    def pack2(key, d1pad):
        mats = []
        for p in ps:
            (w1, b1), (w2, b2) = p[key]
            d = w1.shape[0]
            w1p = jnp.zeros((d1pad, w1.shape[1]), f32).at[:d, :].set(w1)
            w2p = jnp.zeros((d1pad, w2.shape[1]), f32).at[:w2.shape[0], :].set(w2)
            b1p = jnp.broadcast_to(b1, (d1pad, w1.shape[1]))
            b2p = jnp.broadcast_to(b2, (d1pad, w2.shape[1]))
            mats.append(jnp.stack([w1p, b1p, w2p, b2p]))
        return jnp.stack(mats)

    iw = pack2('init_enc', 64)
    gw = pack2('geo_enc', 32)
    ww = pack2('wall_enc', 32)

    def split_conv(key):
        wds, wss, bds = [], [], []
        for p in ps:
            (w1, b1), _ = p[key]
            wds.append(w1[:160] - w1[160:])
            wss.append(w1[160:])
            bds.append(jnp.broadcast_to(b1, (160, 64)))
        return (jnp.stack(wds + bds), jnp.stack(wss + bds))

    dw1, sw1 = split_conv('mlp1')
    dw2, sw2 = split_conv('mlp2')

    def blk(key):
        w = jnp.zeros((128, 128), f32)
        w = w.at[:64, :64].set(ps[0][key][1][0])
        w = w.at[64:, 64:].set(ps[1][key][1][0])
        b = jnp.concatenate([ps[0][key][1][1], ps[1][key][1][1]])
        return w, b.reshape(1, 128)

    w2blk1, b2cat1 = blk('mlp1')
    w2blk2, b2cat2 = blk('mlp2')

    tws, fws = [], []
    for p in ps:
        (w3, b3), (w4, b4) = p['tail']
        tws.append(w3)
        fws.append(jnp.zeros((64, 8), f32).at[:, 0].set(w4[:, 0]))
    tw = jnp.stack(tws + [jnp.broadcast_to(b3i, (160, 64))
                          for b3i in [ps[0]['tail'][0][1], ps[1]['tail'][0][1]]])
    fw = jnp.stack(fws + [jnp.zeros((64, 8), f32).at[:, 0].set(p['tail'][1][1][0])
                          for p in ps])

    ww_in = wall_batch.astype(f32)

    a1, b1arr, cond = _t0(in10, geo8, ww_in, cat3, bat3,
                          ew, cw, iw, gw, ww, dw1, sw1)
    p1 = _s1(a1, b1arr, src, dst)
    m1 = _t1(p1, w2blk1, b2cat1)
    o1 = _s2(m1, dst, neg)
    a2, b2arr = _tmid(o1, cond, dw2, sw2)
    p2 = _s1(a2, b2arr, src, dst)
    m2 = _t1(p2, w2blk2, b2cat2)
    o2 = _s2(m2, dst, neg)
    q1p, q2p = _ttail(o2, cond, tw, fw)
    return (q1p[:, :1], q2p[:, :1])


# speedup vs baseline: 2.6699x; 1.0766x over previous
import functools
import jax
import jax.numpy as jnp
from jax import lax
from jax.experimental import pallas as pl
from jax.experimental.pallas import tpu as pltpu
from jax.experimental.pallas import tpu_sc as plsc

N = 10000
E = 320000
NW = 32
TN = 1000
TE = 4000
S1_C = 80
S2_C = 2000

f32 = jnp.float32


def _dot(a, b):
    return jax.lax.dot_general(a, b, (((1,), (0,)), ((), ())),
                               preferred_element_type=f32)


def _t0_body(in10_ref, geo_ref, wall_ref, cat_ref, bat_ref,
             ew_ref, cw_ref, iw_ref, gw_ref, ww_ref, dw_ref, sw_ref,
             a_ref, b_ref, cond_ref):
    in10 = in10_ref[...]
    geo = geo_ref[...]
    wb = wall_ref[...]
    cat = cat_ref[0, 0, :]
    bat = bat_ref[0, 0, :]

    cat_oh = (cat[:, None] ==
              lax.broadcasted_iota(jnp.int32, (TN, 10), 1)).astype(f32)
    bat_oh = (bat[:, None] ==
              lax.broadcasted_iota(jnp.int32, (TN, 100), 1)).astype(f32)

    for net in range(2):
        emb = ew_ref[net]
        cW, cb = cw_ref[net, 0], cw_ref[net, 1]
        ctab = _dot(jnp.tanh(emb), cW)
        cf = jnp.tanh(_dot(cat_oh, ctab) + cb[0:1, :])

        w1, b1 = ww_ref[net, 0, 0:1, :], ww_ref[net, 1, 0:1, :]
        w2, b2 = ww_ref[net, 2, :, :], ww_ref[net, 3, 0:1, :]
        wtab = jnp.tanh(_dot(jnp.tanh(wb * w1 + b1), w2) + b2)
        wf = _dot(bat_oh, wtab)

        g1, gb1 = gw_ref[net, 0, 0:8, :], gw_ref[net, 1, 0:1, :]
        g2, gb2 = gw_ref[net, 2, :, :], gw_ref[net, 3, 0:1, :]
        gf = jnp.tanh(_dot(jnp.tanh(_dot(geo, g1) + gb1), g2) + gb2)

        i1, ib1 = iw_ref[net, 0, 0:16, :], iw_ref[net, 1, 0:1, :]
        i2, ib2 = iw_ref[net, 2, :, :], iw_ref[net, 3, 0:1, :]
        h0 = jnp.tanh(_dot(jnp.tanh(_dot(in10, i1) + ib1), i2) + ib2)

        cond = jnp.concatenate([cf, wf, gf], axis=-1)
        g1cat = jnp.concatenate([h0, cond], axis=-1)

        wd, bd = dw_ref[net, :, :], dw_ref[net + 2, 0:1, 0:64]
        ws = sw_ref[net, :, :]
        a_ref[:, net * 64:(net + 1) * 64] = _dot(g1cat, wd) + bd
        b_ref[:, net * 64:(net + 1) * 64] = _dot(g1cat, ws)
        cond_ref[:, net * 96:(net + 1) * 96] = cond


def _t0(in10, geo, wall, cat3, bat3, ew, cw, iw, gw, ww, dw, sw):
    g = N // TN
    full = lambda s: pl.BlockSpec(s, lambda i: tuple(0 for _ in s))
    return pl.pallas_call(
        _t0_body,
        grid=(g,),
        in_specs=[
            pl.BlockSpec((TN, 16), lambda i: (i, 0)),
            pl.BlockSpec((TN, 8), lambda i: (i, 0)),
            full((100, 1)),
            pl.BlockSpec((1, 1, TN), lambda i: (i, 0, 0)),
            pl.BlockSpec((1, 1, TN), lambda i: (i, 0, 0)),
            full(ew.shape), full(cw.shape), full(iw.shape),
            full(gw.shape), full(ww.shape), full(dw.shape), full(sw.shape),
        ],
        out_specs=[
            pl.BlockSpec((TN, 128), lambda i: (i, 0)),
            pl.BlockSpec((TN, 128), lambda i: (i, 0)),
            pl.BlockSpec((TN, 192), lambda i: (i, 0)),
        ],
        out_shape=[
            jax.ShapeDtypeStruct((N, 128), f32),
            jax.ShapeDtypeStruct((N, 128), f32),
            jax.ShapeDtypeStruct((N, 192), f32),
        ],
    )(in10, geo, wall, cat3, bat3, ew, cw, iw, gw, ww, dw, sw)


def _s1_body(a_hbm, b_hbm, src_hbm, dst_hbm, p_hbm,
             dstv0, srcv0, ga0, gb0, dstv1, srcv1, ga1, gb1,
             sg0, sg1, sw0, sw1):
    wid = lax.axis_index("s") * 2 + lax.axis_index("c")
    per_w = E // NW
    nch = per_w // S1_C
    slots = ((dstv0, srcv0, ga0, gb0, sg0, sw0),
             (dstv1, srcv1, ga1, gb1, sg1, sw1))

    def fetch(c, b):
        dv, sv, ga, gb, sg, _ = slots[b]
        base = pl.multiple_of(wid * per_w + c * S1_C, 16)
        pltpu.sync_copy(dst_hbm.at[pl.ds(base, S1_C)], dv)
        pltpu.sync_copy(src_hbm.at[pl.ds(base, S1_C)], sv)
        pltpu.async_copy(a_hbm.at[dv], ga, sg)
        pltpu.async_copy(b_hbm.at[sv], gb, sg)

    def drain_g(b):
        _, _, ga, gb, sg, _ = slots[b]
        pltpu.make_async_copy(a_hbm.at[pl.ds(0, S1_C)], ga, sg).wait()
        pltpu.make_async_copy(b_hbm.at[pl.ds(0, S1_C)], gb, sg).wait()

    def process(c, b):
        _, _, ga, gb, _, sw = slots[b]
        drain_g(b)

        def addrow(r, _):
            for cg in range(8):
                s = pl.ds(cg * 16, 16)
                ga[r, s] = ga[r, s] + gb[r, s]
            return 0
        lax.fori_loop(0, S1_C, addrow, 0)
        base = pl.multiple_of(wid * per_w + c * S1_C, 16)
        pltpu.async_copy(ga, p_hbm.at[pl.ds(base, S1_C), :], sw)

    def wait_w(b):
        _, _, ga, _, _, sw = slots[b]
        pltpu.make_async_copy(ga, p_hbm.at[pl.ds(0, S1_C), :], sw).wait()

    fetch(0, 0)

    def pair(i, _):
        c = 2 * i

        @pl.when(c + 1 < nch)
        def _():
            @pl.when(c > 0)
            def _():
                wait_w(1)
            fetch(c + 1, 1)
        process(c, 0)

        @pl.when(c + 2 < nch)
        def _():
            wait_w(0)
            fetch(c + 2, 0)

        @pl.when(c + 1 < nch)
        def _():
            process(c + 1, 1)
        return 0
    lax.fori_loop(0, (nch + 1) // 2, pair, 0)
    wait_w(0)

    @pl.when(nch > 1)
    def _():
        wait_w(1)


def _s1(a, b, src, dst):
    mesh = plsc.VectorSubcoreMesh(core_axis_name="c", subcore_axis_name="s")
    k = functools.partial(
        pl.kernel, mesh=mesh,
        compiler_params=pltpu.CompilerParams(use_tc_tiling_on_sc=False, needs_layout_passes=False),
        out_type=jax.ShapeDtypeStruct((E, 128), f32),
        scratch_types=[
            pltpu.VMEM((S1_C,), jnp.int32),
            pltpu.VMEM((S1_C,), jnp.int32),
            pltpu.VMEM((S1_C, 128), f32),
            pltpu.VMEM((S1_C, 128), f32),
            pltpu.VMEM((S1_C,), jnp.int32),
            pltpu.VMEM((S1_C,), jnp.int32),
            pltpu.VMEM((S1_C, 128), f32),
            pltpu.VMEM((S1_C, 128), f32),
            pltpu.SemaphoreType.DMA,
            pltpu.SemaphoreType.DMA,
            pltpu.SemaphoreType.DMA,
            pltpu.SemaphoreType.DMA,
        ],
    )(_s1_body)
    return k(a, b, src, dst)


def _t1_body(p_ref, w_ref, b_ref, m_ref):
    t = jnp.tanh(p_ref[...])
    m_ref[...] = _dot(t, w_ref[...]) + b_ref[0:1, :]


def _t1(p, w2blk, b2cat):
    g = E // TE
    return pl.pallas_call(
        _t1_body,
        grid=(g,),
        in_specs=[
            pl.BlockSpec((TE, 128), lambda i: (i, 0)),
            pl.BlockSpec((128, 128), lambda i: (0, 0)),
            pl.BlockSpec((1, 128), lambda i: (0, 0)),
        ],
        out_specs=pl.BlockSpec((TE, 128), lambda i: (i, 0)),
        out_shape=jax.ShapeDtypeStruct((E, 128), f32),
    )(p, w2blk, b2cat)


def _s2_body(m_hbm, dst_hbm, neg_hbm, o_hbm, acc, mbuf, dbuf):
    wid = lax.axis_index("s") * 2 + lax.axis_index("c")
    cg = wid % 16
    h = wid // 16
    half = E // 2
    pltpu.sync_copy(neg_hbm, acc)
    i16 = lax.iota(jnp.int32, 16)
    hi = i16 >> 3
    col = i16 & 7

    def chunk(k, _):
        base = pl.multiple_of(h * half + k * S2_C, 16)
        pltpu.sync_copy(dst_hbm.at[pl.ds(base, S2_C)], dbuf)
        pltpu.sync_copy(m_hbm.at[pl.ds(base, S2_C), pl.ds(cg * 8, 8)], mbuf)

        def pair(p, _):
            ri = 2 * p + hi
            dv = plsc.load_gather(dbuf, [ri])
            dw = plsc.load_gather(dbuf, [ri ^ 1])
            mv = plsc.load_gather(mbuf, [ri, col])
            mw = plsc.load_gather(mbuf, [ri ^ 1, col])
            mv = jnp.where(dv == dw, jnp.maximum(mv, mw), mv)
            old = plsc.load_gather(acc, [dv, col])
            plsc.store_scatter(acc, [dv, col], jnp.maximum(old, mv))
            return 0
        lax.fori_loop(0, S2_C // 2, pair, 0)
        return 0
    lax.fori_loop(0, half // S2_C, chunk, 0)
    pltpu.sync_copy(acc, o_hbm.at[wid])


def _s2(m, dst, neg):
    mesh = plsc.VectorSubcoreMesh(core_axis_name="c", subcore_axis_name="s")
    k = functools.partial(
        pl.kernel, mesh=mesh,
        compiler_params=pltpu.CompilerParams(use_tc_tiling_on_sc=False, needs_layout_passes=False),
        out_type=jax.ShapeDtypeStruct((NW, N, 8), f32),
        scratch_types=[
            pltpu.VMEM((N, 8), f32),
            pltpu.VMEM((S2_C, 8), f32),
            pltpu.VMEM((S2_C,), jnp.int32),
        ],
    )(_s2_body)
    return k(m, dst, neg)


def _strips(o):
    out = []
    for cg in range(16):
        v = jnp.maximum(o[cg], o[cg + 16])
        v = jnp.where(jnp.isfinite(v), v, 0.0)
        out.append(jnp.tanh(v))
    return out


def _tmid_body(o_ref, cond_ref, dw_ref, sw_ref, a_ref, b_ref):
    s = _strips(o_ref[...])
    cond = cond_ref[...]
    for net in range(2):
        cn = cond[:, net * 96:(net + 1) * 96]
        wd, bd = dw_ref[net, :, :], dw_ref[net + 2, 0:1, 0:64]
        ws = sw_ref[net, :, :]
        a = _dot(cn, wd[64:160, :]) + bd
        b = _dot(cn, ws[64:160, :])
        for j in range(8):
            st = s[net * 8 + j]
            a = a + _dot(st, wd[8 * j:8 * j + 8, :])
            b = b + _dot(st, ws[8 * j:8 * j + 8, :])
        a_ref[:, net * 64:(net + 1) * 64] = a
        b_ref[:, net * 64:(net + 1) * 64] = b


def _tmid(o, cond, dw, sw):
    g = N // TN
    return pl.pallas_call(
        _tmid_body,
        grid=(g,),
        in_specs=[
            pl.BlockSpec((NW, TN, 8), lambda i: (0, i, 0)),
            pl.BlockSpec((TN, 192), lambda i: (i, 0)),
            pl.BlockSpec(dw.shape, lambda i: (0, 0, 0)),
            pl.BlockSpec(sw.shape, lambda i: (0, 0, 0)),
        ],
        out_specs=[
            pl.BlockSpec((TN, 128), lambda i: (i, 0)),
            pl.BlockSpec((TN, 128), lambda i: (i, 0)),
        ],
        out_shape=[
            jax.ShapeDtypeStruct((N, 128), f32),
            jax.ShapeDtypeStruct((N, 128), f32),
        ],
    )(o, cond, dw, sw)


def _ttail_body(o_ref, cond_ref, tw_ref, fw_ref, q1_ref, q2_ref):
    s = _strips(o_ref[...])
    cond = cond_ref[...]
    for net in range(2):
        cn = cond[:, net * 96:(net + 1) * 96]
        w3, b3 = tw_ref[net, :, :], tw_ref[net + 2, 0:1, 0:64]
        t = _dot(cn, w3[64:160, :]) + b3
        for j in range(8):
            t = t + _dot(s[net * 8 + j], w3[8 * j:8 * j + 8, :])
        t = jnp.tanh(t)
        w4, b4 = fw_ref[net, :, :], fw_ref[net + 2, 0:1, :]
        q = _dot(t, w4) + b4
        if net == 0:
            q1_ref[...] = q
        else:
            q2_ref[...] = q


def _ttail(o, cond, tw, fw):
    g = N // TN
    return pl.pallas_call(
        _ttail_body,
        grid=(g,),
        in_specs=[
            pl.BlockSpec((NW, TN, 8), lambda i: (0, i, 0)),
            pl.BlockSpec((TN, 192), lambda i: (i, 0)),
            pl.BlockSpec(tw.shape, lambda i: (0, 0, 0)),
            pl.BlockSpec(fw.shape, lambda i: (0, 0, 0)),
        ],
        out_specs=[
            pl.BlockSpec((TN, 8), lambda i: (i, 0)),
            pl.BlockSpec((TN, 8), lambda i: (i, 0)),
        ],
        out_shape=[
            jax.ShapeDtypeStruct((N, 8), f32),
            jax.ShapeDtypeStruct((N, 8), f32),
        ],
    )(o, cond, tw, fw)


def kernel(x, actions, tar_scores, geo, wall_batch, category, batch,
           edge_index, params1, params2):
    in10 = jnp.concatenate([x, actions, tar_scores], axis=-1)
    in10 = jnp.pad(in10, ((0, 0), (0, 6)))
    geo8 = jnp.pad(geo.astype(f32), ((0, 0), (0, 6)))
    cat3 = category.astype(jnp.int32).reshape(N // TN, 1, TN)
    bat3 = batch.astype(jnp.int32).reshape(N // TN, 1, TN)
    src = edge_index[0].astype(jnp.int32)
    dst = edge_index[1].astype(jnp.int32)
    neg = jnp.full((N, 8), -jnp.inf, f32)

    ps = (params1, params2)
    ew = jnp.stack([p['embed_table'] for p in ps])
    cw = jnp.stack([jnp.stack([p['embed_lin'][0],
                               jnp.broadcast_to(p['embed_lin'][1], (32, 32))])
                    for p in ps])

    def pack2(key, d1pad):
        mats = []
        for p in ps:
            (w1, b1), (w2, b2) = p[key]
            d = w1.shape[0]
            w1p = jnp.zeros((d1pad, w1.shape[1]), f32).at[:d, :].set(w1)
            w2p = jnp.zeros((d1pad, w2.shape[1]), f32).at[:w2.shape[0], :].set(w2)
            b1p = jnp.broadcast_to(b1, (d1pad, w1.shape[1]))
            b2p = jnp.broadcast_to(b2, (d1pad, w2.shape[1]))
            mats.append(jnp.stack([w1p, b1p, w2p, b2p]))
        return jnp.stack(mats)

    iw = pack2('init_enc', 64)
    gw = pack2('geo_enc', 32)
    ww = pack2('wall_enc', 32)

    def split_conv(key):
        wds, wss, bds = [], [], []
        for p in ps:
            (w1, b1), _ = p[key]
            wds.append(w1[:160] - w1[160:])
            wss.append(w1[160:])
            bds.append(jnp.broadcast_to(b1, (160, 64)))
        return (jnp.stack(wds + bds), jnp.stack(wss + bds))

    dw1, sw1 = split_conv('mlp1')
    dw2, sw2 = split_conv('mlp2')

    def blk(key):
        w = jnp.zeros((128, 128), f32)
        w = w.at[:64, :64].set(ps[0][key][1][0])
        w = w.at[64:, 64:].set(ps[1][key][1][0])
        b = jnp.concatenate([ps[0][key][1][1], ps[1][key][1][1]])
        return w, b.reshape(1, 128)

    w2blk1, b2cat1 = blk('mlp1')
    w2blk2, b2cat2 = blk('mlp2')

    tws, fws = [], []
    for p in ps:
        (w3, b3), (w4, b4) = p['tail']
        tws.append(w3)
        fws.append(jnp.zeros((64, 8), f32).at[:, 0].set(w4[:, 0]))
    tw = jnp.stack(tws + [jnp.broadcast_to(b3i, (160, 64))
                          for b3i in [ps[0]['tail'][0][1], ps[1]['tail'][0][1]]])
    fw = jnp.stack(fws + [jnp.zeros((64, 8), f32).at[:, 0].set(p['tail'][1][1][0])
                          for p in ps])

    ww_in = wall_batch.astype(f32)

    a1, b1arr, cond = _t0(in10, geo8, ww_in, cat3, bat3,
                          ew, cw, iw, gw, ww, dw1, sw1)
    p1 = _s1(a1, b1arr, src, dst)
    m1 = _t1(p1, w2blk1, b2cat1)
    o1 = _s2(m1, dst, neg)
    a2, b2arr = _tmid(o1, cond, dw2, sw2)
    p2 = _s1(a2, b2arr, src, dst)
    m2 = _t1(p2, w2blk2, b2cat2)
    o2 = _s2(m2, dst, neg)
    q1p, q2p = _ttail(o2, cond, tw, fw)
    return (q1p[:, :1], q2p[:, :1])
